# baseline (speedup 1.0000x reference)
import jax
import jax.numpy as jnp
from jax import lax
from jax.experimental import pallas as pl
from jax.experimental.pallas import tpu as pltpu
from jax.experimental.pallas import tpu_sc as plsc

_N = 10000
_E = 320000
_D = 128
_NC = 2
_NS = 16
_NW = _NC * _NS
_NP = 10240
_CHUNK = 64
_NCHUNKS = 160
_EPWP = _CHUNK * _NCHUNKS
_STRIPE = _NP // _NS
_HSTRIPE = 2 * _NP // _NS
_BN = 1024
_GRID = _NP // _BN
_PAD_NODE = _NP - 1
_NCH2 = _NCHUNKS // 2


def _fill2d(ref, nrow, ncol, value):
    vecs_per_row = ncol // 16

    def body(i, carry):
        r = lax.div(i, vecs_per_row)
        o = lax.rem(i, vecs_per_row) * 16
        ref[r, pl.ds(o, 16)] = jnp.full((16,), value, jnp.float32)
        return carry

    lax.fori_loop(0, nrow * vecs_per_row, body, 0)


def _fill1d(ref, n, value):
    def body(i, carry):
        ref[pl.ds(i * 16, 16)] = jnp.full((16,), value, jnp.float32)
        return carry

    lax.fori_loop(0, n // 16, body, 0)


def _deg_body(src_hbm, dstb_hbm, out_hbm, idx_v, ones_v, buf_v, hist_s, sem):
    c = lax.axis_index("c")
    s = lax.axis_index("s")
    w = s * _NC + c
    _fill1d(ones_v, _CHUNK, 1.0)
    _fill1d(buf_v, _HSTRIPE, 0.0)
    base = pl.multiple_of(s * _HSTRIPE, _HSTRIPE)
    pltpu.sync_copy(buf_v, hist_s.at[pl.ds(base, _HSTRIPE)])
    pltpu.sync_copy(src_hbm.at[w], idx_v)
    plsc.subcore_barrier()

    def body_src(j, carry):
        pltpu.sync_copy(ones_v, hist_s.at[idx_v.at[j]], add=True)
        return carry

    lax.fori_loop(0, _NCHUNKS, body_src, 0)
    pltpu.sync_copy(dstb_hbm.at[w], idx_v)

    def body_dst(j, carry):
        pltpu.sync_copy(ones_v, hist_s.at[idx_v.at[j]], add=True)
        return carry

    lax.fori_loop(0, _NCHUNKS, body_dst, 0)
    plsc.subcore_barrier()
    obase = pl.multiple_of(c * 2 * _NP + s * _HSTRIPE, _HSTRIPE)
    pltpu.sync_copy(hist_s.at[pl.ds(base, _HSTRIPE)], buf_v)
    pltpu.sync_copy(buf_v, out_hbm.at[pl.ds(obase, _HSTRIPE)])


def _agg_body(src_hbm, dst_hbm, h_hbm, out_hbm, sidx, didx, rows_v, agg_s,
              gsem, ssem):
    c = lax.axis_index("c")
    s = lax.axis_index("s")
    w = s * _NC + c

    def zfill(i, carry):
        r = lax.div(i, _D // 16)
        o = lax.rem(i, _D // 16) * 16
        rows_v[0, r, pl.ds(o, 16)] = jnp.zeros((16,), jnp.float32)
        return carry

    lax.fori_loop(0, _CHUNK * (_D // 16), zfill, 0)
    base = pl.multiple_of(s * _STRIPE, _STRIPE)

    def zbody(k, carry):
        pltpu.sync_copy(rows_v.at[0], agg_s.at[pl.ds(base + k * _CHUNK, _CHUNK), :])
        return carry

    lax.fori_loop(0, _STRIPE // _CHUNK, zbody, 0)
    plsc.subcore_barrier()

    for h in range(2):
        pltpu.sync_copy(src_hbm.at[w, pl.ds(h * _NCH2, _NCH2)], sidx)
        pltpu.sync_copy(dst_hbm.at[w, pl.ds(h * _NCH2, _NCH2)], didx)
        pltpu.async_copy(h_hbm.at[sidx.at[0]], rows_v.at[0], gsem)

        def body(j, carry):
            a = lax.rem(j, 2)
            jn = jnp.minimum(j + 1, _NCH2 - 1)
            jp = jnp.maximum(j - 1, 0)
            pltpu.make_async_copy(h_hbm.at[sidx.at[j]], rows_v.at[a],
                                  gsem).wait()

            @pl.when(j >= 1)
            def _():
                pltpu.make_async_copy(rows_v.at[1 - a],
                                      agg_s.at[didx.at[jp]], ssem).wait()

            @pl.when(j + 1 < _NCH2)
            def _():
                pltpu.async_copy(h_hbm.at[sidx.at[jn]], rows_v.at[1 - a], gsem)

            pltpu.async_copy(rows_v.at[a], agg_s.at[didx.at[j]], ssem,
                             add=True)
            return carry

        lax.fori_loop(0, _NCH2, body, 0)
        last = (_NCH2 - 1) % 2
        pltpu.make_async_copy(rows_v.at[last],
                              agg_s.at[didx.at[_NCH2 - 1]], ssem).wait()
    plsc.subcore_barrier()

    def wbody(k, carry):
        pltpu.sync_copy(agg_s.at[pl.ds(base + k * _CHUNK, _CHUNK), :], rows_v.at[0])
        pltpu.sync_copy(rows_v.at[0], out_hbm.at[c, pl.ds(base + k * _CHUNK, _CHUNK), :])
        return carry

    lax.fori_loop(0, _STRIPE // _CHUNK, wbody, 0)


def _norms(degt):
    no = lax.rsqrt(degt[:, 0:1] + degt[:, 2:3] + 1.0)
    ni = lax.rsqrt(degt[:, 1:2] + degt[:, 3:4] + 1.0)
    return no, ni


def _prep_body(x_ref, degt_ref, w0_ref, h0_ref):
    no, _ = _norms(degt_ref[...])
    h0_ref[...] = jnp.dot(x_ref[...] * no, w0_ref[...],
                          preferred_element_type=jnp.float32)


def _mid_body(p_ref, g_ref, degt_ref, b_ref, w_ref, out_ref):
    no, ni = _norms(degt_ref[...])
    agg = p_ref[0] + p_ref[1] + g_ref[...]
    y = jnp.maximum(agg * ni + b_ref[...], 0.0)
    out_ref[...] = jnp.dot(y * no, w_ref[...],
                           preferred_element_type=jnp.float32)


def _fin_body(p_ref, g_ref, degt_ref, b_ref, wfc_ref, bfc_ref, out_ref):
    _, ni = _norms(degt_ref[...])
    agg = p_ref[0] + p_ref[1] + g_ref[...]
    y = jnp.maximum(agg * ni + b_ref[...], 0.0)
    out_ref[...] = jnp.dot(y, wfc_ref[...],
                           preferred_element_type=jnp.float32) + bfc_ref[...]


def _make_calls():
    mesh = plsc.VectorSubcoreMesh(core_axis_name="c", subcore_axis_name="s",
                                  num_cores=_NC, num_subcores=_NS)
    deg_call = pl.kernel(
        _deg_body,
        out_type=jax.ShapeDtypeStruct((_NC * 2 * _NP,), jnp.float32),
        mesh=mesh,
        scratch_types=[
            pltpu.VMEM((_NCHUNKS, _CHUNK), jnp.int32),
            pltpu.VMEM((_CHUNK,), jnp.float32),
            pltpu.VMEM((_HSTRIPE,), jnp.float32),
            pltpu.VMEM_SHARED((2 * _NP,), jnp.float32),
            pltpu.SemaphoreType.DMA,
        ],
    )
    agg_call = pl.kernel(
        _agg_body,
        out_type=jax.ShapeDtypeStruct((_NC, _NP, _D), jnp.float32),
        mesh=mesh,
        scratch_types=[
            pltpu.VMEM((_NCH2, _CHUNK), jnp.int32),
            pltpu.VMEM((_NCH2, _CHUNK), jnp.int32),
            pltpu.VMEM((2, _CHUNK, _D), jnp.float32),
            pltpu.VMEM_SHARED((_NP, _D), jnp.float32),
            pltpu.SemaphoreType.DMA,
            pltpu.SemaphoreType.DMA,
        ],
    )

    full = lambda shape: pl.BlockSpec(shape, lambda i: tuple(0 for _ in shape))
    row_blk = pl.BlockSpec((_BN, _D), lambda i: (i, 0))
    deg_blk = pl.BlockSpec((_BN, 4), lambda i: (i, 0))
    prep_call = pl.pallas_call(
        _prep_body,
        grid=(_GRID,),
        in_specs=[row_blk, deg_blk, full((_D, _D))],
        out_specs=row_blk,
        out_shape=jax.ShapeDtypeStruct((_NP, _D), jnp.float32),
    )
    mid_call = pl.pallas_call(
        _mid_body,
        grid=(_GRID,),
        in_specs=[
            pl.BlockSpec((_NC, _BN, _D), lambda i: (0, i, 0)),
            row_blk,
            deg_blk,
            full((1, _D)),
            full((_D, _D)),
        ],
        out_specs=row_blk,
        out_shape=jax.ShapeDtypeStruct((_NP, _D), jnp.float32),
    )
    fin_call = pl.pallas_call(
        _fin_body,
        grid=(_GRID,),
        in_specs=[
            pl.BlockSpec((_NC, _BN, _D), lambda i: (0, i, 0)),
            row_blk,
            deg_blk,
            full((1, _D)),
            full((_D, _D)),
            full((1, _D)),
        ],
        out_specs=row_blk,
        out_shape=jax.ShapeDtypeStruct((_NP, _D), jnp.float32),
    )
    return deg_call, agg_call, prep_call, mid_call, fin_call


def kernel(features, edge_index, W0, b0, W1, b1, W2, b2, W_fc, b_fc):
    deg_call, agg_call, prep_call, mid_call, fin_call = _make_calls()

    pad_e = _NW * _EPWP - _E
    src = jnp.concatenate(
        [edge_index[0].reshape(_NW, _E // _NW),
         jnp.full((_NW, pad_e // _NW), _PAD_NODE, jnp.int32)], axis=1
    ).reshape(_NW, _NCHUNKS, _CHUNK)
    dst = jnp.concatenate(
        [edge_index[1].reshape(_NW, _E // _NW),
         jnp.full((_NW, pad_e // _NW), _PAD_NODE, jnp.int32)], axis=1
    ).reshape(_NW, _NCHUNKS, _CHUNK)
    dstb = dst + _NP

    xp = jnp.pad(features, ((0, _NP - _N), (0, 0)))

    degf = deg_call(src, dstb)
    degt = degf.reshape(4, _NP).T

    h0 = prep_call(xp, degt, W0)
    p = agg_call(src, dst, h0)
    h1 = mid_call(p, h0, degt, b0.reshape(1, _D), W1)
    p = agg_call(src, dst, h1)
    h2 = mid_call(p, h1, degt, b1.reshape(1, _D), W2)
    p = agg_call(src, dst, h2)
    out = fin_call(p, h2, degt, b2.reshape(1, _D), W_fc, b_fc.reshape(1, _D))
    return out[:_N]


# speedup vs baseline: 1.0979x; 1.0979x over previous
import jax
import jax.numpy as jnp
from jax import lax
from jax.experimental import pallas as pl
from jax.experimental.pallas import tpu as pltpu
from jax.experimental.pallas import tpu_sc as plsc

_N = 10000
_E = 320000
_D = 128
_NC = 2
_NS = 16
_NW = _NC * _NS
_NP = 10240
_CHUNK = 128
_NCHUNKS = 80
_EPWP = _CHUNK * _NCHUNKS
_STRIPE = _NP // _NS
_HSTRIPE = 2 * _NP // _NS
_BN = 1024
_GRID = _NP // _BN
_PAD_NODE = _NP - 1
_NCH2 = _NCHUNKS // 2


def _fill2d(ref, nrow, ncol, value):
    vecs_per_row = ncol // 16

    def body(i, carry):
        r = lax.div(i, vecs_per_row)
        o = lax.rem(i, vecs_per_row) * 16
        ref[r, pl.ds(o, 16)] = jnp.full((16,), value, jnp.float32)
        return carry

    lax.fori_loop(0, nrow * vecs_per_row, body, 0)


def _fill1d(ref, n, value):
    def body(i, carry):
        ref[pl.ds(i * 16, 16)] = jnp.full((16,), value, jnp.float32)
        return carry

    lax.fori_loop(0, n // 16, body, 0)


def _deg_body(src_hbm, dstb_hbm, out_hbm, idx_v, ones_v, buf_v, hist_s, sem):
    c = lax.axis_index("c")
    s = lax.axis_index("s")
    w = s * _NC + c
    _fill1d(ones_v, _CHUNK, 1.0)
    _fill1d(buf_v, _HSTRIPE, 0.0)
    base = pl.multiple_of(s * _HSTRIPE, _HSTRIPE)
    pltpu.sync_copy(buf_v, hist_s.at[pl.ds(base, _HSTRIPE)])
    pltpu.sync_copy(src_hbm.at[w], idx_v)
    plsc.subcore_barrier()

    def body_src(j, carry):
        pltpu.sync_copy(ones_v, hist_s.at[idx_v.at[j]], add=True)
        return carry

    lax.fori_loop(0, _NCHUNKS, body_src, 0)
    pltpu.sync_copy(dstb_hbm.at[w], idx_v)

    def body_dst(j, carry):
        pltpu.sync_copy(ones_v, hist_s.at[idx_v.at[j]], add=True)
        return carry

    lax.fori_loop(0, _NCHUNKS, body_dst, 0)
    plsc.subcore_barrier()
    obase = pl.multiple_of(c * 2 * _NP + s * _HSTRIPE, _HSTRIPE)
    pltpu.sync_copy(hist_s.at[pl.ds(base, _HSTRIPE)], buf_v)
    pltpu.sync_copy(buf_v, out_hbm.at[pl.ds(obase, _HSTRIPE)])


def _agg_body(src_hbm, dst_hbm, h_hbm, out_hbm, sidx, didx, rows_v, agg_s,
              gsem, ssem):
    c = lax.axis_index("c")
    s = lax.axis_index("s")
    w = s * _NC + c

    def zfill(i, carry):
        r = lax.div(i, _D // 16)
        o = lax.rem(i, _D // 16) * 16
        rows_v[0, r, pl.ds(o, 16)] = jnp.zeros((16,), jnp.float32)
        return carry

    lax.fori_loop(0, _CHUNK * (_D // 16), zfill, 0)
    base = pl.multiple_of(s * _STRIPE, _STRIPE)

    def zbody(k, carry):
        pltpu.sync_copy(rows_v.at[0], agg_s.at[pl.ds(base + k * _CHUNK, _CHUNK), :])
        return carry

    lax.fori_loop(0, _STRIPE // _CHUNK, zbody, 0)
    plsc.subcore_barrier()

    for h in range(2):
        pltpu.sync_copy(src_hbm.at[w, pl.ds(h * _NCH2, _NCH2)], sidx)
        pltpu.sync_copy(dst_hbm.at[w, pl.ds(h * _NCH2, _NCH2)], didx)
        pltpu.async_copy(h_hbm.at[sidx.at[0]], rows_v.at[0], gsem)

        def body(j, carry):
            a = lax.rem(j, 2)
            jn = jnp.minimum(j + 1, _NCH2 - 1)
            jp = jnp.maximum(j - 1, 0)
            pltpu.make_async_copy(h_hbm.at[sidx.at[j]], rows_v.at[a],
                                  gsem).wait()

            @pl.when(j >= 1)
            def _():
                pltpu.make_async_copy(rows_v.at[1 - a],
                                      agg_s.at[didx.at[jp]], ssem).wait()

            @pl.when(j + 1 < _NCH2)
            def _():
                pltpu.async_copy(h_hbm.at[sidx.at[jn]], rows_v.at[1 - a], gsem)

            pltpu.async_copy(rows_v.at[a], agg_s.at[didx.at[j]], ssem,
                             add=True)
            return carry

        lax.fori_loop(0, _NCH2, body, 0)
        last = (_NCH2 - 1) % 2
        pltpu.make_async_copy(rows_v.at[last],
                              agg_s.at[didx.at[_NCH2 - 1]], ssem).wait()
    plsc.subcore_barrier()

    def wbody(k, carry):
        pltpu.sync_copy(agg_s.at[pl.ds(base + k * _CHUNK, _CHUNK), :], rows_v.at[0])
        pltpu.sync_copy(rows_v.at[0], out_hbm.at[c, pl.ds(base + k * _CHUNK, _CHUNK), :])
        return carry

    lax.fori_loop(0, _STRIPE // _CHUNK, wbody, 0)


def _norms(degt):
    no = lax.rsqrt(degt[:, 0:1] + degt[:, 2:3] + 1.0)
    ni = lax.rsqrt(degt[:, 1:2] + degt[:, 3:4] + 1.0)
    return no, ni


def _prep_body(x_ref, degt_ref, w0_ref, h0_ref):
    no, _ = _norms(degt_ref[...])
    h0_ref[...] = jnp.dot(x_ref[...] * no, w0_ref[...],
                          preferred_element_type=jnp.float32)


def _mid_body(p_ref, g_ref, degt_ref, b_ref, w_ref, out_ref):
    no, ni = _norms(degt_ref[...])
    agg = p_ref[0] + p_ref[1] + g_ref[...]
    y = jnp.maximum(agg * ni + b_ref[...], 0.0)
    out_ref[...] = jnp.dot(y * no, w_ref[...],
                           preferred_element_type=jnp.float32)


def _fin_body(p_ref, g_ref, degt_ref, b_ref, wfc_ref, bfc_ref, out_ref):
    _, ni = _norms(degt_ref[...])
    agg = p_ref[0] + p_ref[1] + g_ref[...]
    y = jnp.maximum(agg * ni + b_ref[...], 0.0)
    out_ref[...] = jnp.dot(y, wfc_ref[...],
                           preferred_element_type=jnp.float32) + bfc_ref[...]


def _make_calls():
    mesh = plsc.VectorSubcoreMesh(core_axis_name="c", subcore_axis_name="s",
                                  num_cores=_NC, num_subcores=_NS)
    deg_call = pl.kernel(
        _deg_body,
        out_type=jax.ShapeDtypeStruct((_NC * 2 * _NP,), jnp.float32),
        mesh=mesh,
        scratch_types=[
            pltpu.VMEM((_NCHUNKS, _CHUNK), jnp.int32),
            pltpu.VMEM((_CHUNK,), jnp.float32),
            pltpu.VMEM((_HSTRIPE,), jnp.float32),
            pltpu.VMEM_SHARED((2 * _NP,), jnp.float32),
            pltpu.SemaphoreType.DMA,
        ],
    )
    agg_call = pl.kernel(
        _agg_body,
        out_type=jax.ShapeDtypeStruct((_NC, _NP, _D), jnp.float32),
        mesh=mesh,
        scratch_types=[
            pltpu.VMEM((_NCH2, _CHUNK), jnp.int32),
            pltpu.VMEM((_NCH2, _CHUNK), jnp.int32),
            pltpu.VMEM((2, _CHUNK, _D), jnp.float32),
            pltpu.VMEM_SHARED((_NP, _D), jnp.float32),
            pltpu.SemaphoreType.DMA,
            pltpu.SemaphoreType.DMA,
        ],
    )

    full = lambda shape: pl.BlockSpec(shape, lambda i: tuple(0 for _ in shape))
    row_blk = pl.BlockSpec((_BN, _D), lambda i: (i, 0))
    deg_blk = pl.BlockSpec((_BN, 4), lambda i: (i, 0))
    prep_call = pl.pallas_call(
        _prep_body,
        grid=(_GRID,),
        in_specs=[row_blk, deg_blk, full((_D, _D))],
        out_specs=row_blk,
        out_shape=jax.ShapeDtypeStruct((_NP, _D), jnp.float32),
    )
    mid_call = pl.pallas_call(
        _mid_body,
        grid=(_GRID,),
        in_specs=[
            pl.BlockSpec((_NC, _BN, _D), lambda i: (0, i, 0)),
            row_blk,
            deg_blk,
            full((1, _D)),
            full((_D, _D)),
        ],
        out_specs=row_blk,
        out_shape=jax.ShapeDtypeStruct((_NP, _D), jnp.float32),
    )
    fin_call = pl.pallas_call(
        _fin_body,
        grid=(_GRID,),
        in_specs=[
            pl.BlockSpec((_NC, _BN, _D), lambda i: (0, i, 0)),
            row_blk,
            deg_blk,
            full((1, _D)),
            full((_D, _D)),
            full((1, _D)),
        ],
        out_specs=row_blk,
        out_shape=jax.ShapeDtypeStruct((_NP, _D), jnp.float32),
    )
    return deg_call, agg_call, prep_call, mid_call, fin_call


def kernel(features, edge_index, W0, b0, W1, b1, W2, b2, W_fc, b_fc):
    deg_call, agg_call, prep_call, mid_call, fin_call = _make_calls()

    pad_e = _NW * _EPWP - _E
    src = jnp.concatenate(
        [edge_index[0].reshape(_NW, _E // _NW),
         jnp.full((_NW, pad_e // _NW), _PAD_NODE, jnp.int32)], axis=1
    ).reshape(_NW, _NCHUNKS, _CHUNK)
    dst = jnp.concatenate(
        [edge_index[1].reshape(_NW, _E // _NW),
         jnp.full((_NW, pad_e // _NW), _PAD_NODE, jnp.int32)], axis=1
    ).reshape(_NW, _NCHUNKS, _CHUNK)
    dstb = dst + _NP

    xp = jnp.pad(features, ((0, _NP - _N), (0, 0)))

    degf = deg_call(src, dstb)
    degt = degf.reshape(4, _NP).T

    h0 = prep_call(xp, degt, W0)
    p = agg_call(src, dst, h0)
    h1 = mid_call(p, h0, degt, b0.reshape(1, _D), W1)
    p = agg_call(src, dst, h1)
    h2 = mid_call(p, h1, degt, b1.reshape(1, _D), W2)
    p = agg_call(src, dst, h2)
    out = fin_call(p, h2, degt, b2.reshape(1, _D), W_fc, b_fc.reshape(1, _D))
    return out[:_N]
